# initial kernel scaffold (unmeasured)
import jax
import jax.numpy as jnp
from jax import lax
from jax.experimental import pallas as pl
from jax.experimental.pallas import tpu as pltpu

N_DEV = 4
B_LOC = 2
SQ = 512
SKV = 512
H_TOT = 32
HG = 8
DH = 64
D_MODEL = 768
D_HID = 512
BLK = 64


def kernel(x, Wq, K_ext, V_ext, Wo):
    i = lax.axis_index("i")
    K_loc = lax.dynamic_slice_in_dim(K_ext, i * B_LOC, B_LOC, axis=0)
    V_loc = lax.dynamic_slice_in_dim(V_ext, i * B_LOC, B_LOC, axis=0)
    K_loc = jnp.transpose(K_loc, (0, 2, 1, 3)).reshape(B_LOC * H_TOT, SKV, DH)
    V_loc = jnp.transpose(V_loc, (0, 2, 1, 3)).reshape(B_LOC * H_TOT, SKV, DH)
    W_pack = jnp.stack([Wq, Wo.T])

    def body(x_ref, w_ref, k_ref, v_ref, out_ref,
             buf, ctx_ref, send_sems, recv_sems):
        my = lax.axis_index("i")
        left = lax.rem(my + N_DEV - 1, N_DEV)
        right = lax.rem(my + 1, N_DEV)

        barrier_sem = pltpu.get_barrier_semaphore()
        for nbr in (left, right):
            pl.semaphore_signal(barrier_sem, inc=1, device_id=(nbr,),
                                device_id_type=pl.DeviceIdType.MESH)
        pl.semaphore_wait(barrier_sem, 2)

        buf[0] = w_ref[...]

        rowb = lax.broadcasted_iota(jnp.int32, (SQ, SKV), 0) // BLK
        colb = lax.broadcasted_iota(jnp.int32, (SQ, SKV), 1) // BLK
        mask = (rowb == colb) | (colb == 0) | (lax.rem(rowb + colb, 3) == 0)

        for h in range(N_DEV):
            rdma = None
            if h < N_DEV - 1:
                rdma = pltpu.make_async_remote_copy(
                    src_ref=buf.at[h],
                    dst_ref=buf.at[h + 1],
                    send_sem=send_sems.at[h],
                    recv_sem=recv_sems.at[h],
                    device_id=(right,),
                    device_id_type=pl.DeviceIdType.MESH,
                )
                rdma.start()

            j = lax.rem(my - h + N_DEV, N_DEV)
            wq = buf[h, 0]
            wot = buf[h, 1]
            for b in range(B_LOC):
                qb = jax.lax.dot(x_ref[b], wq,
                                 preferred_element_type=jnp.float32)
                for hh in range(HG):
                    head = b * H_TOT + j * HG + hh
                    q = qb[:, hh * DH:(hh + 1) * DH]
                    k = k_ref[head]
                    s = lax.dot_general(
                        q, k, (((1,), (1,)), ((), ())),
                        preferred_element_type=jnp.float32) * 0.125
                    s = jnp.where(mask, s, -1e9)
                    m = jnp.max(s, axis=1, keepdims=True)
                    w = jnp.exp(s - m)
                    w = w / jnp.sum(w, axis=1, keepdims=True)
                    ctx_ref[:, hh * DH:(hh + 1) * DH] = jax.lax.dot(
                        w, v_ref[head], preferred_element_type=jnp.float32)
                part = lax.dot_general(
                    ctx_ref[...], wot, (((1,), (1,)), ((), ())),
                    preferred_element_type=jnp.float32)
                if h == 0:
                    out_ref[b] = part
                else:
                    out_ref[b] = out_ref[b] + part

            if rdma is not None:
                rdma.wait()

    return pl.pallas_call(
        body,
        out_shape=jax.ShapeDtypeStruct((B_LOC, SQ, D_MODEL), jnp.float32),
        in_specs=[pl.BlockSpec(memory_space=pltpu.VMEM)] * 4,
        out_specs=pl.BlockSpec(memory_space=pltpu.VMEM),
        scratch_shapes=[
            pltpu.VMEM((N_DEV, 2, D_MODEL, D_HID), jnp.float32),
            pltpu.VMEM((SQ, D_HID), jnp.float32),
            pltpu.SemaphoreType.DMA((N_DEV - 1,)),
            pltpu.SemaphoreType.DMA((N_DEV - 1,)),
        ],
        compiler_params=pltpu.CompilerParams(collective_id=0),
    )(x, W_pack, K_loc, V_loc)


# baseline (device time: 168962 ns/iter reference)
import jax
import jax.numpy as jnp
from jax import lax
from jax.experimental import pallas as pl
from jax.experimental.pallas import tpu as pltpu

N_DEV = 4
B_LOC = 2
SQ = 512
SKV = 512
H_TOT = 32
HG = 8
DH = 64
D_MODEL = 768
D_HID = 512
BLK = 64


def kernel(x, Wq, K_ext, V_ext, Wo):
    i = lax.axis_index("i")
    K_loc = lax.dynamic_slice_in_dim(K_ext, i * B_LOC, B_LOC, axis=0)
    V_loc = lax.dynamic_slice_in_dim(V_ext, i * B_LOC, B_LOC, axis=0)
    K_loc = jnp.transpose(K_loc, (0, 2, 1, 3)).reshape(B_LOC * H_TOT, SKV, DH)
    V_loc = jnp.transpose(V_loc, (0, 2, 1, 3)).reshape(B_LOC * H_TOT, SKV, DH)
    W_pack = jnp.stack([Wq, Wo.T])

    def body(x_ref, w_ref, k_ref, v_ref, out_ref,
             buf, ctx_ref, send_sems, recv_sems):
        my = lax.axis_index("i")
        left = lax.rem(my + N_DEV - 1, N_DEV)
        right = lax.rem(my + 1, N_DEV)

        barrier_sem = pltpu.get_barrier_semaphore()
        for nbr in (left, right):
            pl.semaphore_signal(barrier_sem, inc=1, device_id=(nbr,),
                                device_id_type=pl.DeviceIdType.MESH)
        pl.semaphore_wait(barrier_sem, 2)

        buf[0] = w_ref[...]

        rowb = lax.broadcasted_iota(jnp.int32, (SQ, SKV), 0) // BLK
        colb = lax.broadcasted_iota(jnp.int32, (SQ, SKV), 1) // BLK
        mask = (rowb == colb) | (colb == 0) | (lax.rem(rowb + colb, 3) == 0)

        for h in range(N_DEV):
            rdma = None
            if h < N_DEV - 1:
                rdma = pltpu.make_async_remote_copy(
                    src_ref=buf.at[h],
                    dst_ref=buf.at[h + 1],
                    send_sem=send_sems.at[h],
                    recv_sem=recv_sems.at[h],
                    device_id=(right,),
                    device_id_type=pl.DeviceIdType.MESH,
                )
                rdma.start()

            j = lax.rem(my - h + N_DEV, N_DEV)
            wq = buf[h, 0]
            wot = buf[h, 1]
            for b in range(B_LOC):
                qb = jax.lax.dot(x_ref[b], wq,
                                 preferred_element_type=jnp.float32)
                for hh in range(HG):
                    head = b * H_TOT + j * HG + hh
                    q = qb[:, hh * DH:(hh + 1) * DH]
                    k = k_ref[head]
                    s = lax.dot_general(
                        q, k, (((1,), (1,)), ((), ())),
                        preferred_element_type=jnp.float32) * 0.125
                    s = jnp.where(mask, s, -1e9)
                    m = jnp.max(s, axis=1, keepdims=True)
                    w = jnp.exp(s - m)
                    w = w / jnp.sum(w, axis=1, keepdims=True)
                    ctx_ref[:, hh * DH:(hh + 1) * DH] = jax.lax.dot(
                        w, v_ref[head], preferred_element_type=jnp.float32)
                part = lax.dot_general(
                    ctx_ref[...], wot, (((1,), (1,)), ((), ())),
                    preferred_element_type=jnp.float32)
                if h == 0:
                    out_ref[b] = part
                else:
                    out_ref[b] = out_ref[b] + part

            if rdma is not None:
                rdma.wait()

    return pl.pallas_call(
        body,
        out_shape=jax.ShapeDtypeStruct((B_LOC, SQ, D_MODEL), jnp.float32),
        in_specs=[pl.BlockSpec(memory_space=pltpu.VMEM)] * 4,
        out_specs=pl.BlockSpec(memory_space=pltpu.VMEM),
        scratch_shapes=[
            pltpu.VMEM((N_DEV, 2, D_MODEL, D_HID), jnp.float32),
            pltpu.VMEM((SQ, D_HID), jnp.float32),
            pltpu.SemaphoreType.DMA((N_DEV - 1,)),
            pltpu.SemaphoreType.DMA((N_DEV - 1,)),
        ],
        compiler_params=pltpu.CompilerParams(
            collective_id=0, vmem_limit_bytes=100 * 1024 * 1024),
    )(x, W_pack, K_loc, V_loc)
